# Initial kernel scaffold; baseline (speedup 1.0000x reference)
import jax
import jax.numpy as jnp
from jax.experimental import pallas as pl


def kernel(x, adj, weight, bias):
    raise NotImplementedError("write your pallas kernel here")


# speedup vs baseline: 2.1034x; 2.1034x over previous
import jax
import jax.numpy as jnp
from jax.experimental import pallas as pl
from jax.experimental.pallas import tpu as pltpu


def _round_up(x, m):
    return ((x + m - 1) // m) * m


def _support_kernel(x_ref, w_ref, s_ref):
    s_ref[...] = jnp.dot(
        x_ref[...], w_ref[...], preferred_element_type=jnp.float32
    ).astype(s_ref.dtype)


def _aggregate_kernel(adj_ref, s_ref, b_ref, out_ref):
    out_ref[...] = (
        jnp.dot(adj_ref[...], s_ref[...], preferred_element_type=jnp.float32)
        + b_ref[...]
    ).astype(out_ref.dtype)


@jax.jit
def _gcn_forward(x, adj, weight, bias):
    N, in_dim = x.shape
    out_dim = weight.shape[1]
    dtype = x.dtype

    TM = 512

    n_pad = _round_up(N, TM)
    in_pad = _round_up(in_dim, 128)
    out_pad = _round_up(out_dim, 128)

    if n_pad != N or in_pad != in_dim:
        x = jnp.zeros((n_pad, in_pad), dtype).at[:N, :in_dim].set(x)
    if n_pad != N:
        adj = jnp.zeros((n_pad, n_pad), dtype).at[:N, :N].set(adj)
    if in_pad != in_dim or out_pad != out_dim:
        weight = (
            jnp.zeros((in_pad, out_pad), dtype)
            .at[:in_dim, :out_dim]
            .set(weight)
        )
    if out_pad != out_dim:
        bias = jnp.zeros((out_pad,), dtype).at[:out_dim].set(bias)
    b2 = bias.reshape(1, out_pad)

    support = pl.pallas_call(
        _support_kernel,
        out_shape=jax.ShapeDtypeStruct((n_pad, out_pad), jnp.float32),
        grid=(n_pad // TM,),
        in_specs=[
            pl.BlockSpec((TM, in_pad), lambda i: (i, 0)),
            pl.BlockSpec((in_pad, out_pad), lambda i: (0, 0)),
        ],
        out_specs=pl.BlockSpec((TM, out_pad), lambda i: (i, 0)),
        compiler_params=pltpu.CompilerParams(
            dimension_semantics=("parallel",),
        ),
    )(x, weight)

    out = pl.pallas_call(
        _aggregate_kernel,
        out_shape=jax.ShapeDtypeStruct((n_pad, out_pad), dtype),
        grid=(n_pad // TM,),
        in_specs=[
            pl.BlockSpec((TM, n_pad), lambda i: (i, 0)),
            pl.BlockSpec((n_pad, out_pad), lambda i: (0, 0)),
            pl.BlockSpec((1, out_pad), lambda i: (0, 0)),
        ],
        out_specs=pl.BlockSpec((TM, out_pad), lambda i: (i, 0)),
        compiler_params=pltpu.CompilerParams(
            dimension_semantics=("parallel",),
            vmem_limit_bytes=100 * 1024 * 1024,
        ),
    )(adj, support, b2)

    if n_pad != N or out_pad != out_dim:
        out = out[:N, :out_dim]
    return out


def kernel(x, adj, weight, bias):
    return _gcn_forward(x, adj, weight, bias)


# speedup vs baseline: 2.5722x; 1.2229x over previous
import jax
import jax.numpy as jnp
from jax.experimental import pallas as pl
from jax.experimental.pallas import tpu as pltpu


def _round_up(x, m):
    return ((x + m - 1) // m) * m


def _gcn_kernel(adj_ref, x_ref, w_ref, b_ref, out_ref, s_ref):
    j = pl.program_id(1)

    @pl.when(j == 0)
    def _():
        s_ref[...] = jnp.dot(
            x_ref[...], w_ref[...], preferred_element_type=jnp.float32
        )

    out_ref[...] = (
        jnp.dot(adj_ref[...], s_ref[...], preferred_element_type=jnp.float32)
        + b_ref[...]
    ).astype(out_ref.dtype)


@jax.jit
def _gcn_forward(x, adj, weight, bias):
    N, in_dim = x.shape
    out_dim = weight.shape[1]
    dtype = x.dtype

    TM = 512
    NCORES = 2

    n_pad = _round_up(N, TM * NCORES)
    in_pad = _round_up(in_dim, 128)
    out_pad = _round_up(out_dim, 128)

    if n_pad != N or in_pad != in_dim:
        x = jnp.zeros((n_pad, in_pad), dtype).at[:N, :in_dim].set(x)
    if n_pad != N:
        adj = jnp.zeros((n_pad, n_pad), dtype).at[:N, :N].set(adj)
    if in_pad != in_dim or out_pad != out_dim:
        weight = (
            jnp.zeros((in_pad, out_pad), dtype)
            .at[:in_dim, :out_dim]
            .set(weight)
        )
    if out_pad != out_dim:
        bias = jnp.zeros((out_pad,), dtype).at[:out_dim].set(bias)
    b2 = bias.reshape(1, out_pad)

    nj = n_pad // (TM * NCORES)

    out = pl.pallas_call(
        _gcn_kernel,
        out_shape=jax.ShapeDtypeStruct((n_pad, out_pad), dtype),
        grid=(NCORES, nj),
        in_specs=[
            pl.BlockSpec((TM, n_pad), lambda i, j: (i * (n_pad // (TM * NCORES)) + j, 0)),
            pl.BlockSpec((n_pad, in_pad), lambda i, j: (0, 0)),
            pl.BlockSpec((in_pad, out_pad), lambda i, j: (0, 0)),
            pl.BlockSpec((1, out_pad), lambda i, j: (0, 0)),
        ],
        out_specs=pl.BlockSpec(
            (TM, out_pad), lambda i, j: (i * (n_pad // (TM * NCORES)) + j, 0)
        ),
        scratch_shapes=[pltpu.VMEM((n_pad, out_pad), jnp.float32)],
        compiler_params=pltpu.CompilerParams(
            dimension_semantics=("parallel", "arbitrary"),
            vmem_limit_bytes=100 * 1024 * 1024,
        ),
    )(adj, x, weight, b2)

    if n_pad != N or out_pad != out_dim:
        out = out[:N, :out_dim]
    return out


def kernel(x, adj, weight, bias):
    return _gcn_forward(x, adj, weight, bias)
